# baseline (speedup 1.0000x reference)
import functools

import jax
import jax.numpy as jnp
from jax import lax
from jax.experimental import pallas as pl
from jax.experimental.pallas import tpu as pltpu
from jax.experimental.pallas import tpu_sc as plsc

B = 16384
D = 32
V = 1000000
RW = 16
WBLK = 4096
NBLK = -(-V // WBLK)
SEG = NBLK * WBLK
NPK = 4 * SEG
SEGL = SEG // RW
NC = 2
NS = 16
NW = NC * NS
BPW = B // NW
GRP = BPW // 16


def _slice8(tab):
    def body(x_ref, *o_refs):
        x = x_ref[...]
        for k in range(8):
            o_refs[k][...] = x[k]

    outs = pl.pallas_call(
        body,
        grid=(4, NBLK),
        in_specs=[pl.BlockSpec((8, WBLK), lambda a, ib: (a, ib))],
        out_specs=[pl.BlockSpec((WBLK,), lambda a, ib: (a * NBLK + ib))
                   for _ in range(8)],
        out_shape=[jax.ShapeDtypeStruct((NPK,), jnp.float32)
                   for _ in range(8)],
    )(jnp.swapaxes(tab, 0, 1))
    return [o.reshape(NPK // RW, RW) for o in outs]


def _fpmc_body(u_hbm, l_hbm, n_hbm, *rest):
    tabs = rest[:32]
    out_hbm = rest[32]
    (u_v, l_v, n_v, pu_v, pl_v, pn_v, ru_v, rl_v, rn_v,
     ui_b, iu_b, li_b, il_b, acc_v, sem) = rest[33:]

    wid = lax.axis_index("s") * NC + lax.axis_index("c")

    pltpu.sync_copy(u_hbm.at[wid], u_v)
    pltpu.sync_copy(l_hbm.at[wid], l_v)
    pltpu.sync_copy(n_hbm.at[wid], n_v)

    def pre(x):
        return (lax.shift_left(lax.shift_right_logical(x, 12), 8)
                + (lax.shift_right_logical(x, 4) & 255))

    def prep(o, carry):
        sl = pl.ds(pl.multiple_of(o * 16, 16), 16)
        pu_v[sl] = pre(u_v[sl])
        pl_v[sl] = pre(l_v[sl])
        pn_v[sl] = pre(n_v[sl])
        acc_v[sl] = jnp.zeros((16,), jnp.float32)
        return carry

    lax.fori_loop(0, GRP, prep, 0)

    iota16 = lax.iota(jnp.int32, 16)
    mask15 = jnp.full((16,), RW - 1, jnp.int32)

    for j in range(D):
        a, k = j >> 3, j & 7
        if k == 0:
            joff = a * SEGL

            def build(o, c, joff=joff):
                sl = pl.ds(pl.multiple_of(o * 16, 16), 16)
                ru_v[sl] = pu_v[sl] + joff
                rl_v[sl] = pl_v[sl] + joff
                rn_v[sl] = pn_v[sl] + joff
                return c

            lax.fori_loop(0, GRP, build, 0)

        copies = [
            pltpu.async_copy(tabs[k].at[ru_v], ui_b, sem),
            pltpu.async_copy(tabs[8 + k].at[rn_v], iu_b, sem),
            pltpu.async_copy(tabs[16 + k].at[rl_v], li_b, sem),
            pltpu.async_copy(tabs[24 + k].at[rn_v], il_b, sem),
        ]
        for cp in copies:
            cp.wait()

        def extract(o, c):
            o16 = pl.multiple_of(o * 16, 16)
            sl = pl.ds(o16, 16)
            rows = o16 + iota16
            cu = u_v[sl] & mask15
            cl = l_v[sl] & mask15
            cn = n_v[sl] & mask15
            va = plsc.load_gather(ui_b, [rows, cu])
            vb = plsc.load_gather(iu_b, [rows, cn])
            vc = plsc.load_gather(il_b, [rows, cn])
            vd = plsc.load_gather(li_b, [rows, cl])
            acc_v[sl] = acc_v[sl] + va * vb + vc * vd
            return c

        lax.fori_loop(0, GRP, extract, 0)

    def sig(o, carry):
        sl = pl.ds(pl.multiple_of(o * 16, 16), 16)
        acc_v[sl] = 1.0 / (1.0 + jnp.exp(-acc_v[sl]))
        return carry

    lax.fori_loop(0, GRP, sig, 0)

    base = pl.multiple_of(wid * BPW, BPW)
    pltpu.sync_copy(acc_v, out_hbm.at[pl.ds(base, BPW)])


_fpmc = functools.partial(
    pl.kernel,
    out_type=jax.ShapeDtypeStruct((B,), jnp.float32),
    mesh=plsc.VectorSubcoreMesh(core_axis_name="c", subcore_axis_name="s"),
    compiler_params=pltpu.CompilerParams(
        needs_layout_passes=False, use_tc_tiling_on_sc=False),
    scratch_types=[
        pltpu.VMEM((BPW,), jnp.int32),
        pltpu.VMEM((BPW,), jnp.int32),
        pltpu.VMEM((BPW,), jnp.int32),
        pltpu.VMEM((BPW,), jnp.int32),
        pltpu.VMEM((BPW,), jnp.int32),
        pltpu.VMEM((BPW,), jnp.int32),
        pltpu.VMEM((BPW,), jnp.int32),
        pltpu.VMEM((BPW,), jnp.int32),
        pltpu.VMEM((BPW,), jnp.int32),
        pltpu.VMEM((BPW, RW), jnp.float32),
        pltpu.VMEM((BPW, RW), jnp.float32),
        pltpu.VMEM((BPW, RW), jnp.float32),
        pltpu.VMEM((BPW, RW), jnp.float32),
        pltpu.VMEM((BPW,), jnp.float32),
        pltpu.SemaphoreType.DMA,
    ],
)(_fpmc_body)


def kernel(user_id, item_last_click, next_item, UI, IU, LI, IL):
    u = user_id.reshape(NW, BPW).astype(jnp.int32)
    l = item_last_click.reshape(NW, BPW).astype(jnp.int32)
    n = next_item.reshape(NW, BPW).astype(jnp.int32)
    packed = _slice8(UI) + _slice8(IU) + _slice8(LI) + _slice8(IL)
    return _fpmc(u, l, n, *packed)


# speedup vs baseline: 1.8661x; 1.8661x over previous
import functools

import jax
import jax.numpy as jnp
from jax import lax
from jax.experimental import pallas as pl
from jax.experimental.pallas import tpu as pltpu
from jax.experimental.pallas import tpu_sc as plsc

B = 16384
D = 32
V = 1000000
WBLK = 4096
NBLK = -(-V // WBLK)
OR = WBLK // 4
NPACK = NBLK * OR
NC = 2
NS = 16
NW = NC * NS
BPW = B // NW
NCHUNK = 4
CH = BPW // NCHUNK
GPC = CH // 16


def _detile(tab):
    def body(x_ref, o_ref):
        xt = x_ref[...].T
        o_ref[...] = jnp.concatenate(
            [xt[c * OR:(c + 1) * OR] for c in range(4)], axis=1)

    return pl.pallas_call(
        body,
        grid=(NBLK,),
        in_specs=[pl.BlockSpec((D, WBLK), lambda b: (0, b))],
        out_specs=pl.BlockSpec((OR, 128), lambda b: (b, 0)),
        out_shape=jax.ShapeDtypeStruct((NPACK, 128), jnp.float32),
    )(jnp.swapaxes(tab, 0, 1))


def _fpmc_body(u_hbm, l_hbm, n_hbm, ui_hbm, iu_hbm, li_hbm, il_hbm, out_hbm,
               u_v, l_v, n_v, ru_v, rl_v, rn_v,
               ui_b, iu_b, li_b, il_b, out_v, sem):
    wid = lax.axis_index("s") * NC + lax.axis_index("c")

    pltpu.sync_copy(u_hbm.at[wid], u_v)
    pltpu.sync_copy(l_hbm.at[wid], l_v)
    pltpu.sync_copy(n_hbm.at[wid], n_v)

    def line(x):
        return (lax.shift_left(lax.shift_right_logical(x, 12), 10)
                + (x & 1023))

    def prep(o, carry):
        c = o // GPC
        sl = pl.ds(pl.multiple_of((o % GPC) * 16, 16), 16)
        ru_v[c, sl] = line(u_v[c, sl])
        rl_v[c, sl] = line(l_v[c, sl])
        rn_v[c, sl] = line(n_v[c, sl])
        return carry

    lax.fori_loop(0, NCHUNK * GPC, prep, 0)

    iota16 = lax.iota(jnp.int32, 16)
    three = jnp.full((16,), 3, jnp.int32)

    for c in range(NCHUNK):
        copies = [
            pltpu.async_copy(ui_hbm.at[ru_v.at[c]], ui_b, sem),
            pltpu.async_copy(iu_hbm.at[rn_v.at[c]], iu_b, sem),
            pltpu.async_copy(li_hbm.at[rl_v.at[c]], li_b, sem),
            pltpu.async_copy(il_hbm.at[rn_v.at[c]], il_b, sem),
        ]
        for cp in copies:
            cp.wait()

        def group(g, carry, c=c):
            o = pl.multiple_of(g * 16, 16)
            sl = pl.ds(o, 16)
            row = o + iota16
            ub = (lax.shift_right_logical(u_v[c, sl], 10) & three) * D
            lb = (lax.shift_right_logical(l_v[c, sl], 10) & three) * D
            nb = (lax.shift_right_logical(n_v[c, sl], 10) & three) * D
            acc = jnp.zeros((16,), jnp.float32)
            for j in range(D):
                a = plsc.load_gather(ui_b, [row, ub + j])
                b = plsc.load_gather(iu_b, [row, nb + j])
                cc = plsc.load_gather(il_b, [row, nb + j])
                dd = plsc.load_gather(li_b, [row, lb + j])
                acc = acc + a * b + cc * dd
            sig = 1.0 / (1.0 + jnp.exp(-acc))
            out_v[pl.ds(pl.multiple_of(c * CH + g * 16, 16), 16)] = sig
            return carry

        lax.fori_loop(0, GPC, group, 0)

    base = pl.multiple_of(wid * BPW, BPW)
    pltpu.sync_copy(out_v, out_hbm.at[pl.ds(base, BPW)])


_fpmc = functools.partial(
    pl.kernel,
    out_type=jax.ShapeDtypeStruct((B,), jnp.float32),
    mesh=plsc.VectorSubcoreMesh(core_axis_name="c", subcore_axis_name="s"),
    compiler_params=pltpu.CompilerParams(
        needs_layout_passes=False, use_tc_tiling_on_sc=True),
    scratch_types=[
        pltpu.VMEM((NCHUNK, CH), jnp.int32),
        pltpu.VMEM((NCHUNK, CH), jnp.int32),
        pltpu.VMEM((NCHUNK, CH), jnp.int32),
        pltpu.VMEM((NCHUNK, CH), jnp.int32),
        pltpu.VMEM((NCHUNK, CH), jnp.int32),
        pltpu.VMEM((NCHUNK, CH), jnp.int32),
        pltpu.VMEM((CH, 128), jnp.float32),
        pltpu.VMEM((CH, 128), jnp.float32),
        pltpu.VMEM((CH, 128), jnp.float32),
        pltpu.VMEM((CH, 128), jnp.float32),
        pltpu.VMEM((BPW,), jnp.float32),
        pltpu.SemaphoreType.DMA,
    ],
)(_fpmc_body)


def kernel(user_id, item_last_click, next_item, UI, IU, LI, IL):
    u = user_id.reshape(NW, NCHUNK, CH).astype(jnp.int32)
    l = item_last_click.reshape(NW, NCHUNK, CH).astype(jnp.int32)
    n = next_item.reshape(NW, NCHUNK, CH).astype(jnp.int32)
    return _fpmc(u, l, n,
                 _detile(UI), _detile(IU), _detile(LI), _detile(IL))
